# Initial kernel scaffold; baseline (speedup 1.0000x reference)
import jax
import jax.numpy as jnp
from jax.experimental import pallas as pl


def kernel(x, Wc, bc, Wb, bb, Wce, bce):
    raise NotImplementedError("write your pallas kernel here")


# speedup vs baseline: 1.3130x; 1.3130x over previous
import functools

import jax
import jax.numpy as jnp
from jax.experimental import pallas as pl
from jax.experimental.pallas import tpu as pltpu

IMG_SIZE = 512.0


def _head_kernel(x_ref, w_ref, b_ref, lab_ref, box_ref, ce_ref, *, tn, w_dim):
    acc = jnp.dot(w_ref[...], x_ref[0], preferred_element_type=jnp.float32)
    acc = acc + b_ref[...]

    lab_ref[0] = acc[0:20]
    ce_ref[0] = acc[24:25]

    d = jnp.exp(acc[20:24])
    j = pl.program_id(1)
    hw = j * tn + jax.lax.broadcasted_iota(jnp.int32, (1, tn), 1)
    stride = IMG_SIZE / w_dim
    cy = ((hw // w_dim).astype(jnp.float32) + 0.5) * stride
    cx = ((hw % w_dim).astype(jnp.float32) + 0.5) * stride
    box_ref[0] = jnp.concatenate(
        [cx - d[0:1], cy - d[1:2], cx + d[2:3], cy + d[3:4]], axis=0)


def kernel(x, Wc, bc, Wb, bb, Wce, bce):
    B, C, H, W = x.shape
    HW = H * W
    TN = 2048
    nclasses = Wc.shape[0]

    xf = x.reshape(B, C, HW)
    Wf = jnp.concatenate([Wc, Wb, Wce], axis=0)
    bf = jnp.concatenate([bc, bb, bce], axis=0)[:, None]

    grid = (B, HW // TN)
    labels, boxes, ctr = pl.pallas_call(
        functools.partial(_head_kernel, tn=TN, w_dim=W),
        grid=grid,
        in_specs=[
            pl.BlockSpec((1, C, TN), lambda i, j: (i, 0, j)),
            pl.BlockSpec((nclasses + 5, C), lambda i, j: (0, 0)),
            pl.BlockSpec((nclasses + 5, 1), lambda i, j: (0, 0)),
        ],
        out_specs=[
            pl.BlockSpec((1, nclasses, TN), lambda i, j: (i, 0, j)),
            pl.BlockSpec((1, 4, TN), lambda i, j: (i, 0, j)),
            pl.BlockSpec((1, 1, TN), lambda i, j: (i, 0, j)),
        ],
        out_shape=[
            jax.ShapeDtypeStruct((B, nclasses, HW), jnp.float32),
            jax.ShapeDtypeStruct((B, 4, HW), jnp.float32),
            jax.ShapeDtypeStruct((B, 1, HW), jnp.float32),
        ],
        compiler_params=pltpu.CompilerParams(
            dimension_semantics=("parallel", "parallel")),
    )(xf, Wf, bf)

    return (labels.reshape(B, nclasses, H, W),
            boxes.reshape(B, 4, H, W),
            ctr.reshape(B, 1, H, W))


# speedup vs baseline: 1.7079x; 1.3008x over previous
import functools

import jax
import jax.numpy as jnp
from jax.experimental import pallas as pl
from jax.experimental.pallas import tpu as pltpu

IMG_SIZE = 512.0


def _head_kernel(x_ref, w_ref, b_ref, lab_ref, box_ref, ce_ref, *, tn, w_dim):
    acc = jnp.dot(w_ref[...], x_ref[0], preferred_element_type=jnp.float32)
    acc = acc + b_ref[...]

    lab_ref[0] = acc[0:20]
    ce_ref[0] = acc[24:25]

    d = jnp.exp(acc[20:24])
    j = pl.program_id(1)
    hw = j * tn + jax.lax.broadcasted_iota(jnp.int32, (1, tn), 1)
    stride = IMG_SIZE / w_dim
    cy = ((hw // w_dim).astype(jnp.float32) + 0.5) * stride
    cx = ((hw % w_dim).astype(jnp.float32) + 0.5) * stride
    box_ref[0] = jnp.concatenate(
        [cx - d[0:1], cy - d[1:2], cx + d[2:3], cy + d[3:4]], axis=0)


def kernel(x, Wc, bc, Wb, bb, Wce, bce):
    B, C, H, W = x.shape
    HW = H * W
    TN = 16384
    nclasses = Wc.shape[0]

    xf = x.reshape(B, C, HW)
    Wf = jnp.concatenate([Wc, Wb, Wce], axis=0)
    bf = jnp.concatenate([bc, bb, bce], axis=0)[:, None]

    grid = (B, HW // TN)
    labels, boxes, ctr = pl.pallas_call(
        functools.partial(_head_kernel, tn=TN, w_dim=W),
        grid=grid,
        in_specs=[
            pl.BlockSpec((1, C, TN), lambda i, j: (i, 0, j)),
            pl.BlockSpec((nclasses + 5, C), lambda i, j: (0, 0)),
            pl.BlockSpec((nclasses + 5, 1), lambda i, j: (0, 0)),
        ],
        out_specs=[
            pl.BlockSpec((1, nclasses, TN), lambda i, j: (i, 0, j)),
            pl.BlockSpec((1, 4, TN), lambda i, j: (i, 0, j)),
            pl.BlockSpec((1, 1, TN), lambda i, j: (i, 0, j)),
        ],
        out_shape=[
            jax.ShapeDtypeStruct((B, nclasses, HW), jnp.float32),
            jax.ShapeDtypeStruct((B, 4, HW), jnp.float32),
            jax.ShapeDtypeStruct((B, 1, HW), jnp.float32),
        ],
        compiler_params=pltpu.CompilerParams(
            dimension_semantics=("parallel", "parallel")),
    )(xf, Wf, bf)

    return (labels.reshape(B, nclasses, H, W),
            boxes.reshape(B, 4, H, W),
            ctr.reshape(B, 1, H, W))
